# baseline (speedup 1.0000x reference)
import functools

import jax
import jax.numpy as jnp
from jax import lax
from jax.experimental import pallas as pl
from jax.experimental.pallas import tpu as pltpu
from jax.experimental.pallas import tpu_sc as plsc

NC, NS = 2, 16
NW = NC * NS
CHUNK = 128
SB = 8
EPS = 1e-5


def _chunks(total, step):
    out = []
    off = 0
    while off < total:
        out.append((off, min(step, total - off)))
        off += step
    return out


def _sc_hist(n_pad, n_chunks):
    rows_per_tile = n_pad // NS
    mesh = plsc.VectorSubcoreMesh(core_axis_name="c", subcore_axis_name="s",
                                  num_cores=NC, num_subcores=NS)

    n_super = n_chunks // SB

    @functools.partial(
        pl.kernel,
        out_type=jax.ShapeDtypeStruct((NC, n_pad, 128), jnp.float32),
        mesh=mesh,
        scratch_types=[
            pltpu.VMEM((2, SB, CHUNK), jnp.int32),
            pltpu.VMEM((CHUNK, 128), jnp.float32),
            pltpu.VMEM((CHUNK, 128), jnp.float32),
            pltpu.VMEM_SHARED((n_pad, 128), jnp.float32),
        ],
    )
    def hist(dst_hbm, out_hbm, dsb, ones_v, zbuf_v, acc_sh):
        c = lax.axis_index("c")
        s = lax.axis_index("s")
        wid = s * NC + c

        def fill(i, carry):
            for j in range(8):
                ones_v[i, pl.ds(j * 16, 16)] = jnp.full((16,), 1.0,
                                                        jnp.float32)
                zbuf_v[i, pl.ds(j * 16, 16)] = jnp.zeros((16,), jnp.float32)
            return carry

        lax.fori_loop(0, CHUNK, fill, 0)

        row0 = s * rows_per_tile
        for off, sz in _chunks(rows_per_tile, CHUNK):
            pltpu.sync_copy(zbuf_v.at[pl.ds(0, sz)],
                            acc_sh.at[pl.ds(row0 + off, sz)])
        plsc.subcore_barrier()

        base = wid * n_chunks
        pltpu.sync_copy(dst_hbm.at[pl.ds(base, SB)], dsb.at[0])

        def sstep(g, carry):
            sl = lax.rem(g, 2)
            sn = lax.rem(g + 1, 2)

            @pl.when(g + 1 < n_super)
            def _():
                pltpu.sync_copy(dst_hbm.at[pl.ds(base + (g + 1) * SB, SB)],
                                dsb.at[sn])

            for k in range(SB):
                pltpu.sync_copy(ones_v, acc_sh.at[dsb.at[sl, k]], add=True)
            return carry

        lax.fori_loop(0, n_super, sstep, 0)
        plsc.subcore_barrier()

        for off, sz in _chunks(rows_per_tile, CHUNK):
            pltpu.sync_copy(acc_sh.at[pl.ds(row0 + off, sz)],
                            zbuf_v.at[pl.ds(0, sz)])
            pltpu.sync_copy(zbuf_v.at[pl.ds(0, sz)],
                            out_hbm.at[c, pl.ds(row0 + off, sz)])

    return hist


def _sc_segsum(n_pad, d, n_chunks):
    rows_per_tile = n_pad // NS
    mesh = plsc.VectorSubcoreMesh(core_axis_name="c", subcore_axis_name="s",
                                  num_cores=NC, num_subcores=NS)

    @functools.partial(
        pl.kernel,
        out_type=jax.ShapeDtypeStruct((NC, n_pad, d), jnp.float32),
        mesh=mesh,
        scratch_types=[
            pltpu.VMEM((2, SB * CHUNK), jnp.int32),
            pltpu.VMEM((n_chunks, CHUNK), jnp.int32),
            pltpu.VMEM((2, CHUNK, d), jnp.float32),
            pltpu.VMEM_SHARED((n_pad, d), jnp.float32),
            pltpu.SemaphoreType.DMA((2,)),
            pltpu.SemaphoreType.DMA((2,)),
        ],
    )
    def segsum(h_hbm, src_hbm, dst_hbm, out_hbm, ssb, dst_all, rows_v,
               acc_sh, sem, ssem):
        c = lax.axis_index("c")
        s = lax.axis_index("s")
        wid = s * NC + c
        n_super = n_chunks // SB

        def zrow(i, carry):
            for j in range(d // 16):
                rows_v[0, i, pl.ds(j * 16, 16)] = jnp.zeros((16,), jnp.float32)
            return carry

        lax.fori_loop(0, CHUNK, zrow, 0)
        pltpu.sync_copy(dst_hbm.at[pl.ds(wid * n_chunks, n_chunks)], dst_all)

        row0 = s * rows_per_tile
        for off, sz in _chunks(rows_per_tile, CHUNK):
            pltpu.sync_copy(rows_v.at[0, pl.ds(0, sz)],
                            acc_sh.at[pl.ds(row0 + off, sz)])
        plsc.subcore_barrier()

        base = wid * n_chunks * CHUNK
        pltpu.sync_copy(src_hbm.at[pl.ds(base, SB * CHUNK)], ssb.at[0])
        pltpu.async_copy(h_hbm.at[ssb.at[0, pl.ds(0, CHUNK)]], rows_v.at[0],
                         sem.at[0])

        def sstep(g, carry):
            sl = lax.rem(g, 2)
            sn = lax.rem(g + 1, 2)

            @pl.when(g + 1 < n_super)
            def _():
                pltpu.sync_copy(
                    src_hbm.at[pl.ds(base + (g + 1) * SB * CHUNK, SB * CHUNK)],
                    ssb.at[sn])

            for k in range(SB):
                b = k % 2
                bn = (k + 1) % 2
                cur = ssb.at[sl, pl.ds(k * CHUNK, CHUNK)]
                pltpu.make_async_copy(h_hbm.at[cur], rows_v.at[b],
                                      sem.at[b]).wait()
                pltpu.async_copy(rows_v.at[b],
                                 acc_sh.at[dst_all.at[g * SB + k]],
                                 ssem.at[b], add=True)
                if k == 0:
                    @pl.when(g > 0)
                    def _():
                        pltpu.make_async_copy(
                            rows_v.at[bn],
                            acc_sh.at[dst_all.at[g * SB + k - 1]],
                            ssem.at[bn]).wait()
                else:
                    pltpu.make_async_copy(
                        rows_v.at[bn],
                        acc_sh.at[dst_all.at[g * SB + k - 1]],
                        ssem.at[bn]).wait()
                if k < SB - 1:
                    nxt = ssb.at[sl, pl.ds((k + 1) * CHUNK, CHUNK)]
                    pltpu.async_copy(h_hbm.at[nxt], rows_v.at[bn],
                                     sem.at[bn])
                else:
                    @pl.when(g + 1 < n_super)
                    def _():
                        nxt = ssb.at[sn, pl.ds(0, CHUNK)]
                        pltpu.async_copy(h_hbm.at[nxt], rows_v.at[bn],
                                         sem.at[bn])
            return carry

        lax.fori_loop(0, n_super, sstep, 0)
        pltpu.make_async_copy(rows_v.at[(n_chunks - 1) % 2],
                              acc_sh.at[dst_all.at[n_chunks - 1]],
                              ssem.at[(n_chunks - 1) % 2]).wait()
        plsc.subcore_barrier()

        for off, sz in _chunks(rows_per_tile, CHUNK):
            pltpu.sync_copy(acc_sh.at[pl.ds(row0 + off, sz)],
                            rows_v.at[0, pl.ds(0, sz)])
            pltpu.sync_copy(rows_v.at[0, pl.ds(0, sz)],
                            out_hbm.at[c, pl.ds(row0 + off, sz)])

    return segsum


def _dinv(hist_ref, n):
    deg = hist_ref[0, :n, 0:1] + hist_ref[1, :n, 0:1] + 1.0
    return lax.rsqrt(deg)


def _tc_pre_body(x_ref, w0_ref, hist_ref, out_ref):
    n = x_ref.shape[0]
    dinv = _dinv(hist_ref, n)
    u = jnp.dot(x_ref[...], w0_ref[...], preferred_element_type=jnp.float32)
    out_ref[...] = u * dinv


def _tc_mid_body(s0_ref, h0p_ref, hist_ref, b0_ref, g0_ref, be0_ref, w1_ref,
                 out_ref):
    n = h0p_ref.shape[0]
    dinv = _dinv(hist_ref, n)
    agg = (s0_ref[0, :n, :] + s0_ref[1, :n, :] + h0p_ref[...]) * dinv \
        + b0_ref[...]
    mean = jnp.mean(agg, axis=0, keepdims=True)
    ctr = agg - mean
    var = jnp.mean(ctr * ctr, axis=0, keepdims=True)
    t = ctr * lax.rsqrt(var + EPS) * g0_ref[...] + be0_ref[...]
    t = jnp.maximum(t, 0.0)
    u = jnp.dot(t, w1_ref[...], preferred_element_type=jnp.float32)
    out_ref[...] = u * dinv


def _tc_post_body(s1_ref, h1p_ref, hist_ref, b1_ref, g1_ref, be1_ref, x_ref,
                  out_ref):
    n = h1p_ref.shape[0]
    dinv = _dinv(hist_ref, n)
    y = (s1_ref[0, :n, :] + s1_ref[1, :n, :] + h1p_ref[...]) * dinv \
        + b1_ref[...] + x_ref[...]
    mean = jnp.mean(y, axis=0, keepdims=True)
    ctr = y - mean
    var = jnp.mean(ctr * ctr, axis=0, keepdims=True)
    t = ctr * lax.rsqrt(var + EPS) * g1_ref[...] + be1_ref[...]
    out_ref[...] = jnp.maximum(t, 0.0)


def kernel(x, edge_index, W0, b0, W1, b1, gamma0, beta0, gamma1, beta1):
    n, d = x.shape
    src = edge_index[0]
    dst = edge_index[1]
    e = src.shape[0]

    n_pad = ((n + 1 + 127) // 128) * 128
    e_per_w = -(-e // (NW * CHUNK * 8)) * CHUNK * 8
    n_chunks = e_per_w // CHUNK
    pad = NW * e_per_w - e
    ar = jnp.arange(pad, dtype=jnp.int32)
    src_p = jnp.concatenate([src, (ar * 89) % n])
    dst_p = jnp.concatenate([dst, n + ar % (n_pad - n)])
    dst2 = dst_p.reshape(NW * n_chunks, CHUNK)

    f32 = jnp.float32
    hist = _sc_hist(n_pad, n_chunks)(dst2)

    b0r, b1r = b0.reshape(1, d), b1.reshape(1, d)
    g0r, g1r = gamma0.reshape(1, d), gamma1.reshape(1, d)
    be0r, be1r = beta0.reshape(1, d), beta1.reshape(1, d)

    nd = jax.ShapeDtypeStruct((n, d), f32)
    h0p = pl.pallas_call(_tc_pre_body, out_shape=nd)(x, W0, hist)

    segsum = _sc_segsum(n_pad, d, n_chunks)
    s0 = segsum(h0p, src_p, dst2)
    h1p = pl.pallas_call(_tc_mid_body, out_shape=nd)(
        s0, h0p, hist, b0r, g0r, be0r, W1)

    s1 = segsum(h1p, src_p, dst2)
    out = pl.pallas_call(_tc_post_body, out_shape=nd)(
        s1, h1p, hist, b1r, g1r, be1r, x)
    return out


# speedup vs baseline: 1.0953x; 1.0953x over previous
import functools

import jax
import jax.numpy as jnp
from jax import lax
from jax.experimental import pallas as pl
from jax.experimental.pallas import tpu as pltpu
from jax.experimental.pallas import tpu_sc as plsc

NC, NS = 2, 16
NW = NC * NS
CHUNK = 128
SB = 8
EPS = 1e-5


def _chunks(total, step):
    out = []
    off = 0
    while off < total:
        out.append((off, min(step, total - off)))
        off += step
    return out


def _sc_hist(n_pad, n_chunks):
    rows_per_tile = n_pad // NS
    mesh = plsc.VectorSubcoreMesh(core_axis_name="c", subcore_axis_name="s",
                                  num_cores=NC, num_subcores=NS)

    n_super = n_chunks // SB

    @functools.partial(
        pl.kernel,
        out_type=jax.ShapeDtypeStruct((NC, n_pad, 128), jnp.float32),
        mesh=mesh,
        scratch_types=[
            pltpu.VMEM((2, SB, CHUNK), jnp.int32),
            pltpu.VMEM((CHUNK, 128), jnp.float32),
            pltpu.VMEM((CHUNK, 128), jnp.float32),
            pltpu.VMEM_SHARED((n_pad, 128), jnp.float32),
        ],
    )
    def hist(dst_hbm, out_hbm, dsb, ones_v, zbuf_v, acc_sh):
        c = lax.axis_index("c")
        s = lax.axis_index("s")
        wid = s * NC + c

        def fill(i, carry):
            for j in range(8):
                ones_v[i, pl.ds(j * 16, 16)] = jnp.full((16,), 1.0,
                                                        jnp.float32)
                zbuf_v[i, pl.ds(j * 16, 16)] = jnp.zeros((16,), jnp.float32)
            return carry

        lax.fori_loop(0, CHUNK, fill, 0)

        row0 = s * rows_per_tile
        for off, sz in _chunks(rows_per_tile, CHUNK):
            pltpu.sync_copy(zbuf_v.at[pl.ds(0, sz)],
                            acc_sh.at[pl.ds(row0 + off, sz)])
        plsc.subcore_barrier()

        base = wid * n_chunks
        pltpu.sync_copy(dst_hbm.at[pl.ds(base, SB)], dsb.at[0])

        def sstep(g, carry):
            sl = lax.rem(g, 2)
            sn = lax.rem(g + 1, 2)

            @pl.when(g + 1 < n_super)
            def _():
                pltpu.sync_copy(dst_hbm.at[pl.ds(base + (g + 1) * SB, SB)],
                                dsb.at[sn])

            for k in range(SB):
                pltpu.sync_copy(ones_v, acc_sh.at[dsb.at[sl, k]], add=True)
            return carry

        lax.fori_loop(0, n_super, sstep, 0)
        plsc.subcore_barrier()

        pltpu.sync_copy(acc_sh.at[pl.ds(row0, rows_per_tile)],
                        out_hbm.at[c, pl.ds(row0, rows_per_tile)])

    return hist


def _sc_segsum(n_pad, d, n_chunks):
    rows_per_tile = n_pad // NS
    mesh = plsc.VectorSubcoreMesh(core_axis_name="c", subcore_axis_name="s",
                                  num_cores=NC, num_subcores=NS)

    @functools.partial(
        pl.kernel,
        out_type=jax.ShapeDtypeStruct((NC, n_pad, d), jnp.float32),
        mesh=mesh,
        scratch_types=[
            pltpu.VMEM((2, SB * CHUNK), jnp.int32),
            pltpu.VMEM((n_chunks, CHUNK), jnp.int32),
            pltpu.VMEM((2, CHUNK, d), jnp.float32),
            pltpu.VMEM_SHARED((n_pad, d), jnp.float32),
            pltpu.SemaphoreType.DMA((2,)),
        ],
    )
    def segsum(h_hbm, src_hbm, dst_hbm, out_hbm, ssb, dst_all, rows_v,
               acc_sh, sem):
        c = lax.axis_index("c")
        s = lax.axis_index("s")
        wid = s * NC + c
        n_super = n_chunks // SB

        def zrow(i, carry):
            for j in range(d // 16):
                rows_v[0, i, pl.ds(j * 16, 16)] = jnp.zeros((16,), jnp.float32)
            return carry

        lax.fori_loop(0, CHUNK, zrow, 0)
        pltpu.sync_copy(dst_hbm.at[pl.ds(wid * n_chunks, n_chunks)], dst_all)

        row0 = s * rows_per_tile
        for off, sz in _chunks(rows_per_tile, CHUNK):
            pltpu.sync_copy(rows_v.at[0, pl.ds(0, sz)],
                            acc_sh.at[pl.ds(row0 + off, sz)])
        plsc.subcore_barrier()

        base = wid * n_chunks * CHUNK
        pltpu.sync_copy(src_hbm.at[pl.ds(base, SB * CHUNK)], ssb.at[0])
        pltpu.async_copy(h_hbm.at[ssb.at[0, pl.ds(0, CHUNK)]], rows_v.at[0],
                         sem.at[0])

        def sstep(g, carry):
            sl = lax.rem(g, 2)
            sn = lax.rem(g + 1, 2)

            @pl.when(g + 1 < n_super)
            def _():
                pltpu.sync_copy(
                    src_hbm.at[pl.ds(base + (g + 1) * SB * CHUNK, SB * CHUNK)],
                    ssb.at[sn])

            for k in range(SB):
                b = k % 2
                bn = (k + 1) % 2
                if k < SB - 1:
                    nxt = ssb.at[sl, pl.ds((k + 1) * CHUNK, CHUNK)]
                    pltpu.async_copy(h_hbm.at[nxt], rows_v.at[bn],
                                     sem.at[bn])
                else:
                    @pl.when(g + 1 < n_super)
                    def _():
                        nxt = ssb.at[sn, pl.ds(0, CHUNK)]
                        pltpu.async_copy(h_hbm.at[nxt], rows_v.at[bn],
                                         sem.at[bn])

                cur = ssb.at[sl, pl.ds(k * CHUNK, CHUNK)]
                pltpu.make_async_copy(h_hbm.at[cur], rows_v.at[b],
                                      sem.at[b]).wait()
                pltpu.sync_copy(rows_v.at[b],
                                acc_sh.at[dst_all.at[g * SB + k]], add=True)
            return carry

        lax.fori_loop(0, n_super, sstep, 0)
        plsc.subcore_barrier()

        pltpu.sync_copy(acc_sh.at[pl.ds(row0, rows_per_tile)],
                        out_hbm.at[c, pl.ds(row0, rows_per_tile)])

    return segsum


def _dinv(hist_ref, n):
    deg = hist_ref[0, :n, 0:1] + hist_ref[1, :n, 0:1] + 1.0
    return lax.rsqrt(deg)


def _tc_pre_body(x_ref, w0_ref, hist_ref, out_ref):
    n = x_ref.shape[0]
    dinv = _dinv(hist_ref, n)
    u = jnp.dot(x_ref[...], w0_ref[...], preferred_element_type=jnp.float32)
    out_ref[...] = u * dinv


def _tc_mid_body(s0_ref, h0p_ref, hist_ref, b0_ref, g0_ref, be0_ref, w1_ref,
                 out_ref):
    n = h0p_ref.shape[0]
    dinv = _dinv(hist_ref, n)
    agg = (s0_ref[0, :n, :] + s0_ref[1, :n, :] + h0p_ref[...]) * dinv \
        + b0_ref[...]
    mean = jnp.mean(agg, axis=0, keepdims=True)
    ctr = agg - mean
    var = jnp.mean(ctr * ctr, axis=0, keepdims=True)
    t = ctr * lax.rsqrt(var + EPS) * g0_ref[...] + be0_ref[...]
    t = jnp.maximum(t, 0.0)
    u = jnp.dot(t, w1_ref[...], preferred_element_type=jnp.float32)
    out_ref[...] = u * dinv


def _tc_post_body(s1_ref, h1p_ref, hist_ref, b1_ref, g1_ref, be1_ref, x_ref,
                  out_ref):
    n = h1p_ref.shape[0]
    dinv = _dinv(hist_ref, n)
    y = (s1_ref[0, :n, :] + s1_ref[1, :n, :] + h1p_ref[...]) * dinv \
        + b1_ref[...] + x_ref[...]
    mean = jnp.mean(y, axis=0, keepdims=True)
    ctr = y - mean
    var = jnp.mean(ctr * ctr, axis=0, keepdims=True)
    t = ctr * lax.rsqrt(var + EPS) * g1_ref[...] + be1_ref[...]
    out_ref[...] = jnp.maximum(t, 0.0)


def kernel(x, edge_index, W0, b0, W1, b1, gamma0, beta0, gamma1, beta1):
    n, d = x.shape
    src = edge_index[0]
    dst = edge_index[1]
    e = src.shape[0]

    n_pad = ((n + 1 + 127) // 128) * 128
    e_per_w = -(-e // (NW * CHUNK * 8)) * CHUNK * 8
    n_chunks = e_per_w // CHUNK
    pad = NW * e_per_w - e
    ar = jnp.arange(pad, dtype=jnp.int32)
    src_p = jnp.concatenate([src, (ar * 89) % n])
    dst_p = jnp.concatenate([dst, n + ar % (n_pad - n)])
    dst2 = dst_p.reshape(NW * n_chunks, CHUNK)

    f32 = jnp.float32
    hist = _sc_hist(n_pad, n_chunks)(dst2)

    b0r, b1r = b0.reshape(1, d), b1.reshape(1, d)
    g0r, g1r = gamma0.reshape(1, d), gamma1.reshape(1, d)
    be0r, be1r = beta0.reshape(1, d), beta1.reshape(1, d)

    nd = jax.ShapeDtypeStruct((n, d), f32)
    h0p = pl.pallas_call(_tc_pre_body, out_shape=nd)(x, W0, hist)

    segsum = _sc_segsum(n_pad, d, n_chunks)
    s0 = segsum(h0p, src_p, dst2)
    h1p = pl.pallas_call(_tc_mid_body, out_shape=nd)(
        s0, h0p, hist, b0r, g0r, be0r, W1)

    s1 = segsum(h1p, src_p, dst2)
    out = pl.pallas_call(_tc_post_body, out_shape=nd)(
        s1, h1p, hist, b1r, g1r, be1r, x)
    return out


# speedup vs baseline: 1.1180x; 1.0207x over previous
import functools

import jax
import jax.numpy as jnp
from jax import lax
from jax.experimental import pallas as pl
from jax.experimental.pallas import tpu as pltpu
from jax.experimental.pallas import tpu_sc as plsc

NC, NS = 2, 16
NW = NC * NS
CHUNK = 128
SB = 16
EPS = 1e-5


def _chunks(total, step):
    out = []
    off = 0
    while off < total:
        out.append((off, min(step, total - off)))
        off += step
    return out


def _sc_hist(n_pad, n_chunks):
    rows_per_tile = n_pad // NS
    mesh = plsc.VectorSubcoreMesh(core_axis_name="c", subcore_axis_name="s",
                                  num_cores=NC, num_subcores=NS)

    n_super = n_chunks // SB

    @functools.partial(
        pl.kernel,
        out_type=jax.ShapeDtypeStruct((NC, n_pad, 128), jnp.float32),
        mesh=mesh,
        scratch_types=[
            pltpu.VMEM((2, SB, CHUNK), jnp.int32),
            pltpu.VMEM((CHUNK, 128), jnp.float32),
            pltpu.VMEM((CHUNK, 128), jnp.float32),
            pltpu.VMEM_SHARED((n_pad, 128), jnp.float32),
        ],
    )
    def hist(dst_hbm, out_hbm, dsb, ones_v, zbuf_v, acc_sh):
        c = lax.axis_index("c")
        s = lax.axis_index("s")
        wid = s * NC + c

        def fill(i, carry):
            for j in range(8):
                ones_v[i, pl.ds(j * 16, 16)] = jnp.full((16,), 1.0,
                                                        jnp.float32)
                zbuf_v[i, pl.ds(j * 16, 16)] = jnp.zeros((16,), jnp.float32)
            return carry

        lax.fori_loop(0, CHUNK, fill, 0)

        row0 = s * rows_per_tile
        for off, sz in _chunks(rows_per_tile, CHUNK):
            pltpu.sync_copy(zbuf_v.at[pl.ds(0, sz)],
                            acc_sh.at[pl.ds(row0 + off, sz)])
        plsc.subcore_barrier()

        base = wid * n_chunks
        pltpu.sync_copy(dst_hbm.at[pl.ds(base, SB)], dsb.at[0])

        def sstep(g, carry):
            sl = lax.rem(g, 2)
            sn = lax.rem(g + 1, 2)

            @pl.when(g + 1 < n_super)
            def _():
                pltpu.sync_copy(dst_hbm.at[pl.ds(base + (g + 1) * SB, SB)],
                                dsb.at[sn])

            for k in range(SB):
                pltpu.sync_copy(ones_v, acc_sh.at[dsb.at[sl, k]], add=True)
            return carry

        lax.fori_loop(0, n_super, sstep, 0)
        plsc.subcore_barrier()

        pltpu.sync_copy(acc_sh.at[pl.ds(row0, rows_per_tile)],
                        out_hbm.at[c, pl.ds(row0, rows_per_tile)])

    return hist


def _sc_segsum(n_pad, d, n_chunks):
    rows_per_tile = n_pad // NS
    mesh = plsc.VectorSubcoreMesh(core_axis_name="c", subcore_axis_name="s",
                                  num_cores=NC, num_subcores=NS)

    @functools.partial(
        pl.kernel,
        out_type=jax.ShapeDtypeStruct((NC, n_pad, d), jnp.float32),
        mesh=mesh,
        scratch_types=[
            pltpu.VMEM((2, SB * CHUNK), jnp.int32),
            pltpu.VMEM((n_chunks, CHUNK), jnp.int32),
            pltpu.VMEM((2, CHUNK, d), jnp.float32),
            pltpu.VMEM_SHARED((n_pad, d), jnp.float32),
            pltpu.SemaphoreType.DMA((2,)),
        ],
    )
    def segsum(h_hbm, src_hbm, dst_hbm, out_hbm, ssb, dst_all, rows_v,
               acc_sh, sem):
        c = lax.axis_index("c")
        s = lax.axis_index("s")
        wid = s * NC + c
        n_super = n_chunks // SB

        def zrow(i, carry):
            for j in range(d // 16):
                rows_v[0, i, pl.ds(j * 16, 16)] = jnp.zeros((16,), jnp.float32)
            return carry

        lax.fori_loop(0, CHUNK, zrow, 0)
        pltpu.sync_copy(dst_hbm.at[pl.ds(wid * n_chunks, n_chunks)], dst_all)

        row0 = s * rows_per_tile
        for off, sz in _chunks(rows_per_tile, CHUNK):
            pltpu.sync_copy(rows_v.at[0, pl.ds(0, sz)],
                            acc_sh.at[pl.ds(row0 + off, sz)])
        plsc.subcore_barrier()

        base = wid * n_chunks * CHUNK
        pltpu.sync_copy(src_hbm.at[pl.ds(base, SB * CHUNK)], ssb.at[0])
        pltpu.async_copy(h_hbm.at[ssb.at[0, pl.ds(0, CHUNK)]], rows_v.at[0],
                         sem.at[0])

        def sstep(g, carry):
            sl = lax.rem(g, 2)
            sn = lax.rem(g + 1, 2)

            @pl.when(g + 1 < n_super)
            def _():
                pltpu.sync_copy(
                    src_hbm.at[pl.ds(base + (g + 1) * SB * CHUNK, SB * CHUNK)],
                    ssb.at[sn])

            for k in range(SB):
                b = k % 2
                bn = (k + 1) % 2
                if k < SB - 1:
                    nxt = ssb.at[sl, pl.ds((k + 1) * CHUNK, CHUNK)]
                    pltpu.async_copy(h_hbm.at[nxt], rows_v.at[bn],
                                     sem.at[bn])
                else:
                    @pl.when(g + 1 < n_super)
                    def _():
                        nxt = ssb.at[sn, pl.ds(0, CHUNK)]
                        pltpu.async_copy(h_hbm.at[nxt], rows_v.at[bn],
                                         sem.at[bn])

                cur = ssb.at[sl, pl.ds(k * CHUNK, CHUNK)]
                pltpu.make_async_copy(h_hbm.at[cur], rows_v.at[b],
                                      sem.at[b]).wait()
                pltpu.sync_copy(rows_v.at[b],
                                acc_sh.at[dst_all.at[g * SB + k]], add=True)
            return carry

        lax.fori_loop(0, n_super, sstep, 0)
        plsc.subcore_barrier()

        pltpu.sync_copy(acc_sh.at[pl.ds(row0, rows_per_tile)],
                        out_hbm.at[c, pl.ds(row0, rows_per_tile)])

    return segsum


def _dinv(hist_ref, n):
    deg = hist_ref[0, :n, 0:1] + hist_ref[1, :n, 0:1] + 1.0
    return lax.rsqrt(deg)


def _tc_pre_body(x_ref, w0_ref, hist_ref, out_ref):
    n = x_ref.shape[0]
    dinv = _dinv(hist_ref, n)
    u = jnp.dot(x_ref[...], w0_ref[...], preferred_element_type=jnp.float32)
    out_ref[...] = u * dinv


def _tc_mid_body(s0_ref, h0p_ref, hist_ref, b0_ref, g0_ref, be0_ref, w1_ref,
                 out_ref):
    n = h0p_ref.shape[0]
    dinv = _dinv(hist_ref, n)
    agg = (s0_ref[0, :n, :] + s0_ref[1, :n, :] + h0p_ref[...]) * dinv \
        + b0_ref[...]
    mean = jnp.mean(agg, axis=0, keepdims=True)
    ctr = agg - mean
    var = jnp.mean(ctr * ctr, axis=0, keepdims=True)
    t = ctr * lax.rsqrt(var + EPS) * g0_ref[...] + be0_ref[...]
    t = jnp.maximum(t, 0.0)
    u = jnp.dot(t, w1_ref[...], preferred_element_type=jnp.float32)
    out_ref[...] = u * dinv


def _tc_post_body(s1_ref, h1p_ref, hist_ref, b1_ref, g1_ref, be1_ref, x_ref,
                  out_ref):
    n = h1p_ref.shape[0]
    dinv = _dinv(hist_ref, n)
    y = (s1_ref[0, :n, :] + s1_ref[1, :n, :] + h1p_ref[...]) * dinv \
        + b1_ref[...] + x_ref[...]
    mean = jnp.mean(y, axis=0, keepdims=True)
    ctr = y - mean
    var = jnp.mean(ctr * ctr, axis=0, keepdims=True)
    t = ctr * lax.rsqrt(var + EPS) * g1_ref[...] + be1_ref[...]
    out_ref[...] = jnp.maximum(t, 0.0)


def kernel(x, edge_index, W0, b0, W1, b1, gamma0, beta0, gamma1, beta1):
    n, d = x.shape
    src = edge_index[0]
    dst = edge_index[1]
    e = src.shape[0]

    n_pad = ((n + 1 + 127) // 128) * 128
    e_per_w = -(-e // (NW * CHUNK * 8)) * CHUNK * 8
    n_chunks = e_per_w // CHUNK
    pad = NW * e_per_w - e
    ar = jnp.arange(pad, dtype=jnp.int32)
    src_p = jnp.concatenate([src, (ar * 89) % n])
    dst_p = jnp.concatenate([dst, n + ar % (n_pad - n)])
    dst2 = dst_p.reshape(NW * n_chunks, CHUNK)

    f32 = jnp.float32
    hist = _sc_hist(n_pad, n_chunks)(dst2)

    b0r, b1r = b0.reshape(1, d), b1.reshape(1, d)
    g0r, g1r = gamma0.reshape(1, d), gamma1.reshape(1, d)
    be0r, be1r = beta0.reshape(1, d), beta1.reshape(1, d)

    nd = jax.ShapeDtypeStruct((n, d), f32)
    h0p = pl.pallas_call(_tc_pre_body, out_shape=nd)(x, W0, hist)

    segsum = _sc_segsum(n_pad, d, n_chunks)
    s0 = segsum(h0p, src_p, dst2)
    h1p = pl.pallas_call(_tc_mid_body, out_shape=nd)(
        s0, h0p, hist, b0r, g0r, be0r, W1)

    s1 = segsum(h1p, src_p, dst2)
    out = pl.pallas_call(_tc_post_body, out_shape=nd)(
        s1, h1p, hist, b1r, g1r, be1r, x)
    return out
